# baseline (device time: 181796 ns/iter reference)
import jax
import jax.numpy as jnp
from jax import lax
from jax.experimental import pallas as pl
from jax.experimental.pallas import tpu as pltpu

N_DEV = 32
B, SQ, SKV, HQ_LOC, DH = 2, 256, 256, 4, 64
D_MODEL = 512
ROWS = B * SQ
CHUNK = ROWS // N_DEV


def kernel(x, Wq, K_ext, V_ext, Wo):
    i = lax.axis_index("i")
    K_loc = lax.dynamic_slice(K_ext, (0, 0, i * HQ_LOC, 0), (B, SKV, HQ_LOC, DH))
    V_loc = lax.dynamic_slice(V_ext, (0, 0, i * HQ_LOC, 0), (B, SKV, HQ_LOC, DH))
    K_loc = jnp.transpose(K_loc, (0, 2, 1, 3))
    V_loc = jnp.transpose(V_loc, (0, 2, 1, 3))
    x2 = x.reshape(ROWS, D_MODEL)

    def body(x_ref, wq_ref, k_ref, v_ref, wo_ref, out_ref,
             ctx_ref, acc_ref, rs_buf, rs_send, rs_recv, ag_send, ag_recv):
        my = lax.axis_index("i")
        left = (my + N_DEV - 1) % N_DEV
        right = (my + 1) % N_DEV

        q = jnp.dot(x_ref[:, :], wq_ref[:, :], preferred_element_type=jnp.float32)
        for b in range(B):
            for h in range(HQ_LOC):
                qbh = q[b * SQ:(b + 1) * SQ, h * DH:(h + 1) * DH]
                s = lax.dot_general(
                    qbh, k_ref[b, h], (((1,), (1,)), ((), ())),
                    preferred_element_type=jnp.float32,
                ) * 0.125
                rb = lax.broadcasted_iota(jnp.int32, (SQ, SKV), 0) // 64
                cb = lax.broadcasted_iota(jnp.int32, (SQ, SKV), 1) // 64
                s = jnp.where(cb <= rb, s, -1e9)
                m = jnp.max(s, axis=1, keepdims=True)
                e = jnp.exp(s - m)
                w = e / jnp.sum(e, axis=1, keepdims=True)
                ctx_ref[b * SQ:(b + 1) * SQ, h * DH:(h + 1) * DH] = jnp.dot(
                    w, v_ref[b, h], preferred_element_type=jnp.float32)
        acc_ref[:, :] = jnp.dot(ctx_ref[:, :], wo_ref[:, :],
                                preferred_element_type=jnp.float32)

        bar = pltpu.get_barrier_semaphore()
        for nbr in (left, right):
            pl.semaphore_signal(bar, inc=1, device_id=(nbr,),
                                device_id_type=pl.DeviceIdType.MESH)
        pl.semaphore_wait(bar, 2)

        for h in range(N_DEV - 1):
            cs = (my + N_DEV - h) % N_DEV
            rdma = pltpu.make_async_remote_copy(
                src_ref=acc_ref.at[pl.ds(cs * CHUNK, CHUNK), :],
                dst_ref=rs_buf.at[h],
                send_sem=rs_send.at[h],
                recv_sem=rs_recv.at[h],
                device_id=(right,),
                device_id_type=pl.DeviceIdType.MESH,
            )
            rdma.start()
            rdma.wait()
            cr = (my + N_DEV - h - 1) % N_DEV
            r0 = cr * CHUNK
            acc_ref[pl.ds(r0, CHUNK), :] = acc_ref[pl.ds(r0, CHUNK), :] + rs_buf[h]

        own = (my + 1) % N_DEV
        out_ref[pl.ds(own * CHUNK, CHUNK), :] = acc_ref[pl.ds(own * CHUNK, CHUNK), :]

        for h in range(N_DEV - 1):
            cs = (my + 1 + N_DEV - h) % N_DEV
            rdma = pltpu.make_async_remote_copy(
                src_ref=out_ref.at[pl.ds(cs * CHUNK, CHUNK), :],
                dst_ref=out_ref.at[pl.ds(cs * CHUNK, CHUNK), :],
                send_sem=ag_send.at[h],
                recv_sem=ag_recv.at[h],
                device_id=(right,),
                device_id_type=pl.DeviceIdType.MESH,
            )
            rdma.start()
            rdma.wait()

    out = pl.pallas_call(
        body,
        out_shape=jax.ShapeDtypeStruct((ROWS, D_MODEL), jnp.float32),
        in_specs=[pl.BlockSpec(memory_space=pltpu.VMEM)] * 5,
        out_specs=pl.BlockSpec(memory_space=pltpu.VMEM),
        scratch_shapes=[
            pltpu.VMEM((ROWS, HQ_LOC * DH), jnp.float32),
            pltpu.VMEM((ROWS, D_MODEL), jnp.float32),
            pltpu.VMEM((N_DEV - 1, CHUNK, D_MODEL), jnp.float32),
            pltpu.SemaphoreType.DMA((N_DEV - 1,)),
            pltpu.SemaphoreType.DMA((N_DEV - 1,)),
            pltpu.SemaphoreType.DMA((N_DEV - 1,)),
            pltpu.SemaphoreType.DMA((N_DEV - 1,)),
        ],
        compiler_params=pltpu.CompilerParams(collective_id=0),
    )(x2, Wq, K_loc, V_loc, Wo)
    return out.reshape(B, SQ, D_MODEL)


# device time: 85362 ns/iter; 2.1297x vs baseline; 2.1297x over previous
import jax
import jax.numpy as jnp
from jax import lax
from jax.experimental import pallas as pl
from jax.experimental.pallas import tpu as pltpu

N_DEV = 32
LOG2 = 5
B, SQ, SKV, HQ_LOC, DH = 2, 256, 256, 4, 64
D_MODEL = 512
ROWS = B * SQ
CHUNK = ROWS // N_DEV

RS_HALF = [ROWS >> (k + 1) for k in range(LOG2)]
RS_OFF = [0]
for _h in RS_HALF[:-1]:
    RS_OFF.append(RS_OFF[-1] + _h)
RS_BUF_ROWS = sum(RS_HALF)


def kernel(x, Wq, K_ext, V_ext, Wo):
    i = lax.axis_index("i")
    K_loc = lax.dynamic_slice(K_ext, (0, 0, i * HQ_LOC, 0), (B, SKV, HQ_LOC, DH))
    V_loc = lax.dynamic_slice(V_ext, (0, 0, i * HQ_LOC, 0), (B, SKV, HQ_LOC, DH))
    K_loc = jnp.transpose(K_loc, (0, 2, 1, 3))
    V_loc = jnp.transpose(V_loc, (0, 2, 1, 3))
    x2 = x.reshape(ROWS, D_MODEL)

    def body(x_ref, wq_ref, k_ref, v_ref, wo_ref, out_ref,
             ctx_ref, acc_ref, rs_buf, rs_send, rs_recv, ag_send, ag_recv):
        my = lax.axis_index("i")

        q = jnp.dot(x_ref[:, :], wq_ref[:, :], preferred_element_type=jnp.float32)
        for b in range(B):
            for h in range(HQ_LOC):
                qbh = q[b * SQ:(b + 1) * SQ, h * DH:(h + 1) * DH]
                s = lax.dot_general(
                    qbh, k_ref[b, h], (((1,), (1,)), ((), ())),
                    preferred_element_type=jnp.float32,
                ) * 0.125
                rb = lax.broadcasted_iota(jnp.int32, (SQ, SKV), 0) // 64
                cb = lax.broadcasted_iota(jnp.int32, (SQ, SKV), 1) // 64
                s = jnp.where(cb <= rb, s, -1e9)
                m = jnp.max(s, axis=1, keepdims=True)
                e = jnp.exp(s - m)
                w = e / jnp.sum(e, axis=1, keepdims=True)
                ctx_ref[b * SQ:(b + 1) * SQ, h * DH:(h + 1) * DH] = jnp.dot(
                    w, v_ref[b, h], preferred_element_type=jnp.float32)
        acc_ref[:, :] = jnp.dot(ctx_ref[:, :], wo_ref[:, :],
                                preferred_element_type=jnp.float32)

        bar = pltpu.get_barrier_semaphore()
        for k in range(LOG2):
            pl.semaphore_signal(bar, inc=1, device_id=(my ^ (1 << k),),
                                device_id_type=pl.DeviceIdType.MESH)
        pl.semaphore_wait(bar, LOG2)

        lo = my * 0
        for k in range(LOG2):
            half = RS_HALF[k]
            bit = (my >> k) & 1
            keep_lo = pl.multiple_of(lo + bit * half, CHUNK)
            send_lo = pl.multiple_of(lo + (1 - bit) * half, CHUNK)
            rdma = pltpu.make_async_remote_copy(
                src_ref=acc_ref.at[pl.ds(send_lo, half), :],
                dst_ref=rs_buf.at[pl.ds(RS_OFF[k], half), :],
                send_sem=rs_send.at[k],
                recv_sem=rs_recv.at[k],
                device_id=(my ^ (1 << k),),
                device_id_type=pl.DeviceIdType.MESH,
            )
            rdma.start()
            rdma.wait()
            acc_ref[pl.ds(keep_lo, half), :] = (
                acc_ref[pl.ds(keep_lo, half), :]
                + rs_buf[pl.ds(RS_OFF[k], half), :]
            )
            lo = keep_lo
        out_ref[pl.ds(lo, CHUNK), :] = acc_ref[pl.ds(lo, CHUNK), :]

        for idx, j in enumerate(range(LOG2 - 1, -1, -1)):
            size = CHUNK << (LOG2 - 1 - j)
            glo = pl.multiple_of(lo & ~(size - 1), size)
            rdma = pltpu.make_async_remote_copy(
                src_ref=out_ref.at[pl.ds(glo, size), :],
                dst_ref=out_ref.at[pl.ds(glo, size), :],
                send_sem=ag_send.at[idx],
                recv_sem=ag_recv.at[idx],
                device_id=(my ^ (1 << j),),
                device_id_type=pl.DeviceIdType.MESH,
            )
            rdma.start()
            rdma.wait()

    out = pl.pallas_call(
        body,
        out_shape=jax.ShapeDtypeStruct((ROWS, D_MODEL), jnp.float32),
        in_specs=[pl.BlockSpec(memory_space=pltpu.VMEM)] * 5,
        out_specs=pl.BlockSpec(memory_space=pltpu.VMEM),
        scratch_shapes=[
            pltpu.VMEM((ROWS, HQ_LOC * DH), jnp.float32),
            pltpu.VMEM((ROWS, D_MODEL), jnp.float32),
            pltpu.VMEM((RS_BUF_ROWS, D_MODEL), jnp.float32),
            pltpu.SemaphoreType.DMA((LOG2,)),
            pltpu.SemaphoreType.DMA((LOG2,)),
            pltpu.SemaphoreType.DMA((LOG2,)),
            pltpu.SemaphoreType.DMA((LOG2,)),
        ],
        compiler_params=pltpu.CompilerParams(collective_id=0),
    )(x2, Wq, K_loc, V_loc, Wo)
    return out.reshape(B, SQ, D_MODEL)
